# baseline (speedup 1.0000x reference)
import jax
import jax.numpy as jnp
from jax.experimental import pallas as pl
from jax.experimental.pallas import tpu as pltpu

_BM = 512
_SUB = 4
_SM = _BM // _SUB


def _matmul_body(x_ref, w_ref, b_ref, o_ref):
    j = pl.program_id(1)
    acc = jax.lax.dot_general(
        x_ref[pl.ds(j * _SM, _SM), :],
        w_ref[...],
        dimension_numbers=(((1,), (1,)), ((), ())),
        preferred_element_type=jnp.float32,
    )
    o_ref[...] = acc + b_ref[...]


@jax.jit
def kernel(input, weight, bias):
    m, k = input.shape
    n = weight.shape[0]
    grid = (m // _BM, _SUB)
    return pl.pallas_call(
        _matmul_body,
        grid=grid,
        in_specs=[
            pl.BlockSpec((_BM, k), lambda i, j: (i, 0)),
            pl.BlockSpec((n, k), lambda i, j: (0, 0)),
            pl.BlockSpec((1, n), lambda i, j: (0, 0)),
        ],
        out_specs=pl.BlockSpec((_SM, n), lambda i, j: (i * _SUB + j, 0)),
        out_shape=jax.ShapeDtypeStruct((m, n), jnp.float32),
        compiler_params=pltpu.CompilerParams(
            dimension_semantics=("arbitrary", "arbitrary"),
        ),
    )(input, weight, bias.reshape(1, n))


# speedup vs baseline: 1.7042x; 1.7042x over previous
import jax
import jax.numpy as jnp
from jax.experimental import pallas as pl
from jax.experimental.pallas import tpu as pltpu

_BM = 256


def _matmul_body(xa_ref, xb_ref, w_ref, b_ref, o_ref):
    wt = w_ref[...]
    bb = b_ref[...]
    o_ref[pl.ds(0, _BM), :] = jax.lax.dot_general(
        xa_ref[...], wt,
        dimension_numbers=(((1,), (1,)), ((), ())),
        preferred_element_type=jnp.float32,
    ) + bb
    o_ref[pl.ds(_BM, _BM), :] = jax.lax.dot_general(
        xb_ref[...], wt,
        dimension_numbers=(((1,), (1,)), ((), ())),
        preferred_element_type=jnp.float32,
    ) + bb


@jax.jit
def kernel(input, weight, bias):
    m, k = input.shape
    n = weight.shape[0]
    grid = (m // (2 * _BM),)
    return pl.pallas_call(
        _matmul_body,
        grid=grid,
        in_specs=[
            pl.BlockSpec((_BM, k), lambda i: (2 * i, 0)),
            pl.BlockSpec((_BM, k), lambda i: (2 * i + 1, 0)),
            pl.BlockSpec((n, k), lambda i: (0, 0)),
            pl.BlockSpec((1, n), lambda i: (0, 0)),
        ],
        out_specs=pl.BlockSpec((2 * _BM, n), lambda i: (i, 0)),
        out_shape=jax.ShapeDtypeStruct((m, n), jnp.float32),
        compiler_params=pltpu.CompilerParams(
            dimension_semantics=("parallel",),
        ),
    )(input, input, weight, bias.reshape(1, n))
